# Initial kernel scaffold; baseline (speedup 1.0000x reference)
import jax
import jax.numpy as jnp
from jax.experimental import pallas as pl


def kernel(hidden_states, sim_matrix, gates):
    raise NotImplementedError("write your pallas kernel here")


# speedup vs baseline: 1.5932x; 1.5932x over previous
import functools

import jax
import jax.numpy as jnp
from jax.experimental import pallas as pl

_NEG_MAX = -3.4028234663852886e38


def _gating_kernel(h_ref, sim_ref, gates_ref, probs_ref, fb_ref, logits_ref,
                   mask_ref, *, k_fallback):
    i = pl.program_id(0)

    aff = jnp.dot(h_ref[...], sim_ref[...], preferred_element_type=jnp.float32)
    logits = aff - jax.nn.sigmoid(gates_ref[...])
    gated = jnp.maximum(logits, 0.0)
    mask_act = (logits > 0.0).astype(jnp.float32)
    num_active = jnp.sum(mask_act, axis=1, keepdims=True)
    inactive = num_active == 0.0

    logits_ref[...] = logits
    mask_ref[...] = mask_act

    masked = jnp.where(mask_act > 0.0, gated, _NEG_MAX)
    m = jnp.max(masked, axis=1, keepdims=True)
    ex = jnp.exp(masked - m)
    probs_ref[...] = ex / jnp.sum(ex, axis=1, keepdims=True)

    @pl.when(i == 0)
    def _init():
        fb_ref[...] = jnp.zeros((1, 1), jnp.float32)

    fb_ref[...] += jnp.sum(inactive.astype(jnp.float32), axis=(0, 1),
                           keepdims=True)

    @pl.when(jnp.any(inactive))
    def _fallback():
        a = logits[:, :, None]
        b = logits[:, None, :]
        gt = (b > a).astype(jnp.float32)
        ii = jax.lax.broadcasted_iota(jnp.int32, gt.shape, 1)
        ff = jax.lax.broadcasted_iota(jnp.int32, gt.shape, 2)
        tie = ((b == a) & (ff < ii)).astype(jnp.float32)
        rank = jnp.sum(gt + tie, axis=2)
        fb_mask = (rank < float(k_fallback)).astype(jnp.float32)
        final_mask = jnp.where(inactive, jnp.maximum(mask_act, fb_mask),
                               mask_act)
        masked2 = jnp.where(final_mask > 0.0, gated, _NEG_MAX)
        m2 = jnp.max(masked2, axis=1, keepdims=True)
        ex2 = jnp.exp(masked2 - m2)
        mask_ref[...] = final_mask
        probs_ref[...] = ex2 / jnp.sum(ex2, axis=1, keepdims=True)


def kernel(hidden_states, sim_matrix, gates):
    B, T, C = hidden_states.shape
    E = sim_matrix.shape[1]
    M = B * T
    TM = 512
    while M % TM:
        TM //= 2
    flat = hidden_states.reshape(M, C)
    gates2 = gates.reshape(1, E)

    f32 = jnp.float32
    probs, fb, logits, mask = pl.pallas_call(
        functools.partial(_gating_kernel, k_fallback=E // 2),
        grid=(M // TM,),
        in_specs=[
            pl.BlockSpec((TM, C), lambda i: (i, 0)),
            pl.BlockSpec((C, E), lambda i: (0, 0)),
            pl.BlockSpec((1, E), lambda i: (0, 0)),
        ],
        out_specs=[
            pl.BlockSpec((TM, E), lambda i: (i, 0)),
            pl.BlockSpec((1, 1), lambda i: (0, 0)),
            pl.BlockSpec((TM, E), lambda i: (i, 0)),
            pl.BlockSpec((TM, E), lambda i: (i, 0)),
        ],
        out_shape=[
            jax.ShapeDtypeStruct((M, E), f32),
            jax.ShapeDtypeStruct((1, 1), f32),
            jax.ShapeDtypeStruct((M, E), f32),
            jax.ShapeDtypeStruct((M, E), f32),
        ],
    )(flat, sim_matrix, gates2)

    return (probs.reshape(B, T, E), fb[0, 0], logits.reshape(B, T, E),
            mask.reshape(B, T, E))


# speedup vs baseline: 1.5954x; 1.0014x over previous
import functools

import jax
import jax.numpy as jnp
from jax.experimental import pallas as pl

_NEG_MAX = -3.4028234663852886e38


def _gating_kernel(h_ref, sim_ref, gates_ref, probs_ref, fb_ref, logits_ref,
                   mask_ref, *, k_fallback):
    i = pl.program_id(0)

    aff = jnp.dot(h_ref[...], sim_ref[...], preferred_element_type=jnp.float32)
    logits = aff - jax.nn.sigmoid(gates_ref[...])
    mx = jnp.max(logits, axis=1, keepdims=True)
    inactive = mx <= 0.0
    mask_act = (logits > 0.0).astype(jnp.float32)
    ex = mask_act * jnp.exp(logits - mx)
    s = jnp.sum(ex, axis=1, keepdims=True)

    logits_ref[...] = logits
    mask_ref[...] = mask_act
    probs_ref[...] = ex * (1.0 / s)

    @pl.when(i == 0)
    def _init():
        fb_ref[...] = jnp.zeros((1, 1), jnp.float32)

    fb_ref[...] += jnp.sum(inactive.astype(jnp.float32), axis=(0, 1),
                           keepdims=True)

    @pl.when(jnp.any(inactive))
    def _fallback():
        a = logits[:, :, None]
        b = logits[:, None, :]
        gt = (b > a).astype(jnp.float32)
        ii = jax.lax.broadcasted_iota(jnp.int32, gt.shape, 1)
        ff = jax.lax.broadcasted_iota(jnp.int32, gt.shape, 2)
        tie = ((b == a) & (ff < ii)).astype(jnp.float32)
        rank = jnp.sum(gt + tie, axis=2)
        fb_mask = (rank < float(k_fallback)).astype(jnp.float32)
        gated = jnp.maximum(logits, 0.0)
        final_mask = jnp.where(inactive, jnp.maximum(mask_act, fb_mask),
                               mask_act)
        masked2 = jnp.where(final_mask > 0.0, gated, _NEG_MAX)
        m2 = jnp.max(masked2, axis=1, keepdims=True)
        ex2 = jnp.exp(masked2 - m2)
        mask_ref[...] = final_mask
        probs_ref[...] = ex2 / jnp.sum(ex2, axis=1, keepdims=True)


def kernel(hidden_states, sim_matrix, gates):
    B, T, C = hidden_states.shape
    E = sim_matrix.shape[1]
    M = B * T
    TM = 512
    while M % TM:
        TM //= 2
    flat = hidden_states.reshape(M, C)
    gates2 = gates.reshape(1, E)

    f32 = jnp.float32
    probs, fb, logits, mask = pl.pallas_call(
        functools.partial(_gating_kernel, k_fallback=E // 2),
        grid=(M // TM,),
        in_specs=[
            pl.BlockSpec((TM, C), lambda i: (i, 0)),
            pl.BlockSpec((C, E), lambda i: (0, 0)),
            pl.BlockSpec((1, E), lambda i: (0, 0)),
        ],
        out_specs=[
            pl.BlockSpec((TM, E), lambda i: (i, 0)),
            pl.BlockSpec((1, 1), lambda i: (0, 0)),
            pl.BlockSpec((TM, E), lambda i: (i, 0)),
            pl.BlockSpec((TM, E), lambda i: (i, 0)),
        ],
        out_shape=[
            jax.ShapeDtypeStruct((M, E), f32),
            jax.ShapeDtypeStruct((1, 1), f32),
            jax.ShapeDtypeStruct((M, E), f32),
            jax.ShapeDtypeStruct((M, E), f32),
        ],
    )(flat, sim_matrix, gates2)

    return (probs.reshape(B, T, E), fb[0, 0], logits.reshape(B, T, E),
            mask.reshape(B, T, E))


# speedup vs baseline: 1.5968x; 1.0009x over previous
import functools

import jax
import jax.numpy as jnp
from jax.experimental import pallas as pl

_NEG_MAX = -3.4028234663852886e38


def _gating_kernel(h_ref, sim_ref, gates_ref, probs_ref, fb_ref, logits_ref,
                   mask_ref, *, k_fallback):
    first = (pl.program_id(0) == 0) & (pl.program_id(1) == 0)

    aff = jnp.dot(h_ref[0], sim_ref[...], preferred_element_type=jnp.float32)
    logits = aff - jax.nn.sigmoid(gates_ref[...])
    mx = jnp.max(logits, axis=1, keepdims=True)
    inactive = mx <= 0.0
    mask_act = (logits > 0.0).astype(jnp.float32)
    ex = mask_act * jnp.exp(logits - mx)
    s = jnp.sum(ex, axis=1, keepdims=True)

    logits_ref[0] = logits
    mask_ref[0] = mask_act
    probs_ref[0] = ex * (1.0 / s)

    @pl.when(first)
    def _init():
        fb_ref[...] = jnp.zeros((1, 1), jnp.float32)

    fb_ref[...] += jnp.sum(inactive.astype(jnp.float32), axis=(0, 1),
                           keepdims=True)

    @pl.when(jnp.any(inactive))
    def _fallback():
        a = logits[:, :, None]
        b = logits[:, None, :]
        gt = (b > a).astype(jnp.float32)
        ii = jax.lax.broadcasted_iota(jnp.int32, gt.shape, 1)
        ff = jax.lax.broadcasted_iota(jnp.int32, gt.shape, 2)
        tie = ((b == a) & (ff < ii)).astype(jnp.float32)
        rank = jnp.sum(gt + tie, axis=2)
        fb_mask = (rank < float(k_fallback)).astype(jnp.float32)
        gated = jnp.maximum(logits, 0.0)
        final_mask = jnp.where(inactive, jnp.maximum(mask_act, fb_mask),
                               mask_act)
        masked2 = jnp.where(final_mask > 0.0, gated, _NEG_MAX)
        m2 = jnp.max(masked2, axis=1, keepdims=True)
        ex2 = jnp.exp(masked2 - m2)
        mask_ref[0] = final_mask
        probs_ref[0] = ex2 / jnp.sum(ex2, axis=1, keepdims=True)


def kernel(hidden_states, sim_matrix, gates):
    B, T, C = hidden_states.shape
    E = sim_matrix.shape[1]
    TM = 512
    while T % TM:
        TM //= 2
    gates2 = gates.reshape(1, E)

    f32 = jnp.float32
    probs, fb, logits, mask = pl.pallas_call(
        functools.partial(_gating_kernel, k_fallback=E // 2),
        grid=(B, T // TM),
        in_specs=[
            pl.BlockSpec((1, TM, C), lambda b, t: (b, t, 0)),
            pl.BlockSpec((C, E), lambda b, t: (0, 0)),
            pl.BlockSpec((1, E), lambda b, t: (0, 0)),
        ],
        out_specs=[
            pl.BlockSpec((1, TM, E), lambda b, t: (b, t, 0)),
            pl.BlockSpec((1, 1), lambda b, t: (0, 0)),
            pl.BlockSpec((1, TM, E), lambda b, t: (b, t, 0)),
            pl.BlockSpec((1, TM, E), lambda b, t: (b, t, 0)),
        ],
        out_shape=[
            jax.ShapeDtypeStruct((B, T, E), f32),
            jax.ShapeDtypeStruct((1, 1), f32),
            jax.ShapeDtypeStruct((B, T, E), f32),
            jax.ShapeDtypeStruct((B, T, E), f32),
        ],
    )(hidden_states, sim_matrix, gates2)

    return (probs, fb[0, 0], logits, mask)


# speedup vs baseline: 2.1078x; 1.3200x over previous
import functools

import jax
import jax.numpy as jnp
from jax.experimental import pallas as pl

_NEG_MAX = -3.4028234663852886e38


def _gating_kernel(h_ref, simt_ref, gates_ref, probs_ref, fb_ref, logits_ref,
                   mask_ref, *, k_fallback):
    first = (pl.program_id(0) == 0) & (pl.program_id(1) == 0)

    aff = jax.lax.dot_general(simt_ref[...], h_ref[0],
                              (((1,), (1,)), ((), ())),
                              preferred_element_type=jnp.float32)
    sig = jnp.transpose(jax.nn.sigmoid(gates_ref[...]))
    logits = aff - sig
    mx = jnp.max(logits, axis=0, keepdims=True)
    inactive = mx <= 0.0
    mask_act = (logits > 0.0).astype(jnp.float32)
    ex = mask_act * jnp.exp(logits - mx)
    s = jnp.sum(ex, axis=0, keepdims=True)

    logits_ref[0] = logits
    mask_ref[0] = mask_act
    probs_ref[0] = ex * (1.0 / s)

    @pl.when(first)
    def _init():
        fb_ref[...] = jnp.zeros((1, 1), jnp.float32)

    fb_ref[...] += jnp.sum(inactive.astype(jnp.float32), axis=(0, 1),
                           keepdims=True)

    @pl.when(jnp.any(inactive))
    def _fallback():
        a = logits[:, None, :]
        b = logits[None, :, :]
        gt = (b > a).astype(jnp.float32)
        ee = jax.lax.broadcasted_iota(jnp.int32, gt.shape, 0)
        ff = jax.lax.broadcasted_iota(jnp.int32, gt.shape, 1)
        tie = ((b == a) & (ff < ee)).astype(jnp.float32)
        rank = jnp.sum(gt + tie, axis=1)
        fb_mask = (rank < float(k_fallback)).astype(jnp.float32)
        gated = jnp.maximum(logits, 0.0)
        final_mask = jnp.where(inactive, jnp.maximum(mask_act, fb_mask),
                               mask_act)
        masked2 = jnp.where(final_mask > 0.0, gated, _NEG_MAX)
        m2 = jnp.max(masked2, axis=0, keepdims=True)
        ex2 = jnp.exp(masked2 - m2)
        mask_ref[0] = final_mask
        probs_ref[0] = ex2 / jnp.sum(ex2, axis=0, keepdims=True)


def kernel(hidden_states, sim_matrix, gates):
    B, T, C = hidden_states.shape
    E = sim_matrix.shape[1]
    TM = 512
    while T % TM:
        TM //= 2
    sim_t = sim_matrix.T
    gates2 = gates.reshape(1, E)

    f32 = jnp.float32
    probs_t, fb, logits_t, mask_t = pl.pallas_call(
        functools.partial(_gating_kernel, k_fallback=E // 2),
        grid=(B, T // TM),
        in_specs=[
            pl.BlockSpec((1, TM, C), lambda b, t: (b, t, 0)),
            pl.BlockSpec((E, C), lambda b, t: (0, 0)),
            pl.BlockSpec((1, E), lambda b, t: (0, 0)),
        ],
        out_specs=[
            pl.BlockSpec((1, E, TM), lambda b, t: (b, 0, t)),
            pl.BlockSpec((1, 1), lambda b, t: (0, 0)),
            pl.BlockSpec((1, E, TM), lambda b, t: (b, 0, t)),
            pl.BlockSpec((1, E, TM), lambda b, t: (b, 0, t)),
        ],
        out_shape=[
            jax.ShapeDtypeStruct((B, E, T), f32),
            jax.ShapeDtypeStruct((1, 1), f32),
            jax.ShapeDtypeStruct((B, E, T), f32),
            jax.ShapeDtypeStruct((B, E, T), f32),
        ],
    )(hidden_states, sim_t, gates2)

    return (jnp.swapaxes(probs_t, 1, 2), fb[0, 0],
            jnp.swapaxes(logits_t, 1, 2), jnp.swapaxes(mask_t, 1, 2))


# speedup vs baseline: 2.1682x; 1.0287x over previous
import functools

import jax
import jax.numpy as jnp
from jax.experimental import pallas as pl

_NEG_MAX = -3.4028234663852886e38


def _gating_kernel(h_ref, simt_ref, gates_ref, probs_ref, fb_ref, logits_ref,
                   mask_ref, *, k_fallback):
    first = (pl.program_id(0) == 0) & (pl.program_id(1) == 0)

    aff = jax.lax.dot_general(simt_ref[...], h_ref[0],
                              (((1,), (1,)), ((), ())),
                              preferred_element_type=jnp.float32)
    sig = jnp.transpose(jax.nn.sigmoid(gates_ref[...]))
    logits = aff - sig
    mx = jnp.max(logits, axis=0, keepdims=True)
    inactive = mx <= 0.0
    mask_act = (logits > 0.0).astype(jnp.float32)
    ex = mask_act * jnp.exp(logits - mx)
    s = jnp.sum(ex, axis=0, keepdims=True)

    logits_ref[0] = logits
    mask_ref[0] = mask_act
    probs_ref[0] = ex * (1.0 / s)

    @pl.when(first)
    def _init():
        fb_ref[...] = jnp.zeros((1, 1), jnp.float32)

    fb_ref[...] += jnp.sum(inactive.astype(jnp.float32), axis=(0, 1),
                           keepdims=True)

    @pl.when(jnp.any(inactive))
    def _fallback():
        a = logits[:, None, :]
        b = logits[None, :, :]
        gt = (b > a).astype(jnp.float32)
        ee = jax.lax.broadcasted_iota(jnp.int32, gt.shape, 0)
        ff = jax.lax.broadcasted_iota(jnp.int32, gt.shape, 1)
        tie = ((b == a) & (ff < ee)).astype(jnp.float32)
        rank = jnp.sum(gt + tie, axis=1)
        fb_mask = (rank < float(k_fallback)).astype(jnp.float32)
        gated = jnp.maximum(logits, 0.0)
        final_mask = jnp.where(inactive, jnp.maximum(mask_act, fb_mask),
                               mask_act)
        masked2 = jnp.where(final_mask > 0.0, gated, _NEG_MAX)
        m2 = jnp.max(masked2, axis=0, keepdims=True)
        ex2 = jnp.exp(masked2 - m2)
        mask_ref[0] = final_mask
        probs_ref[0] = ex2 / jnp.sum(ex2, axis=0, keepdims=True)


def kernel(hidden_states, sim_matrix, gates):
    B, T, C = hidden_states.shape
    E = sim_matrix.shape[1]
    TM = 1024
    while T % TM:
        TM //= 2
    sim_t = sim_matrix.T
    gates2 = gates.reshape(1, E)

    f32 = jnp.float32
    probs_t, fb, logits_t, mask_t = pl.pallas_call(
        functools.partial(_gating_kernel, k_fallback=E // 2),
        grid=(B, T // TM),
        in_specs=[
            pl.BlockSpec((1, TM, C), lambda b, t: (b, t, 0)),
            pl.BlockSpec((E, C), lambda b, t: (0, 0)),
            pl.BlockSpec((1, E), lambda b, t: (0, 0)),
        ],
        out_specs=[
            pl.BlockSpec((1, E, TM), lambda b, t: (b, 0, t)),
            pl.BlockSpec((1, 1), lambda b, t: (0, 0)),
            pl.BlockSpec((1, E, TM), lambda b, t: (b, 0, t)),
            pl.BlockSpec((1, E, TM), lambda b, t: (b, 0, t)),
        ],
        out_shape=[
            jax.ShapeDtypeStruct((B, E, T), f32),
            jax.ShapeDtypeStruct((1, 1), f32),
            jax.ShapeDtypeStruct((B, E, T), f32),
            jax.ShapeDtypeStruct((B, E, T), f32),
        ],
    )(hidden_states, sim_t, gates2)

    return (jnp.swapaxes(probs_t, 1, 2), fb[0, 0],
            jnp.swapaxes(logits_t, 1, 2), jnp.swapaxes(mask_t, 1, 2))
